# baseline (speedup 1.0000x reference)
import functools

import jax
import jax.numpy as jnp
from jax import lax
from jax.experimental import pallas as pl
from jax.experimental.pallas import tpu as pltpu
from jax.experimental.pallas import tpu_sc as plsc

_N = 10000
_E = 320000
_D = 128

_NC = 2
_NS = 16
_NW = _NC * _NS

_EPT = _E // _NW
_N_PAD = 10112
_SLAB = _N_PAD // _NS

_BLK = 1280
_GRID = 8
_NSLAB = 8
_CNT_PAD = _NSLAB * _BLK


def _lrelu(v):
    return jnp.where(v >= 0, v, 0.01 * v)


_sc_mesh = plsc.VectorSubcoreMesh(core_axis_name="c", subcore_axis_name="s")


def _make_sc_agg(chunk, idxblk, nidx, with_counts):
    out_type = [jax.ShapeDtypeStruct((_NC, _N_PAD, _D), jnp.float32)]
    scratch = [
        pltpu.VMEM((idxblk, chunk), jnp.int32),
        pltpu.VMEM((idxblk, chunk), jnp.int32),
        pltpu.VMEM((chunk, _D), jnp.float32),
        pltpu.VMEM((chunk, _D), jnp.float32),
        pltpu.VMEM_SHARED((_N_PAD, _D), jnp.float32),
        pltpu.SemaphoreType.DMA,
        pltpu.SemaphoreType.DMA,
        pltpu.SemaphoreType.DMA,
        pltpu.SemaphoreType.DMA,
    ]
    if with_counts:
        out_type.append(jax.ShapeDtypeStruct((_NSLAB, _NW, _BLK), jnp.float32))
        scratch.insert(4, pltpu.VMEM((_CNT_PAD,), jnp.float32))

    @functools.partial(
        pl.kernel,
        mesh=_sc_mesh,
        compiler_params=pltpu.CompilerParams(needs_layout_passes=False),
        out_type=tuple(out_type),
        scratch_types=scratch,
    )
    def _sc_agg(h_hbm, src_hbm, dst_hbm, *rest):
        if with_counts:
            (agg_out, cnt_out, src_v, dst_v, rows_a, rows_b, cnt_v,
             agg_sh, sem_a, sem_b, sem_s, sem_t) = rest
        else:
            (agg_out, src_v, dst_v, rows_a, rows_b,
             agg_sh, sem_a, sem_b, sem_s, sem_t) = rest
        cid = lax.axis_index("c")
        sid = lax.axis_index("s")
        wid = sid * _NC + cid

        zbase = sid * _SLAB
        def zrow(i, carry):
            for g in range(_D // 16):
                rows_a[i, pl.ds(g * 16, 16)] = jnp.zeros((16,), jnp.float32)
            return carry
        lax.fori_loop(0, chunk, zrow, 0)
        for f in range(_SLAB // chunk):
            pltpu.sync_copy(rows_a, agg_sh.at[pl.ds(zbase + f * chunk, chunk)])
        rem = _SLAB % chunk
        if rem:
            pltpu.sync_copy(
                rows_a.at[pl.ds(0, rem)],
                agg_sh.at[pl.ds(zbase + (_SLAB // chunk) * chunk, rem)])

        if with_counts:
            def zbody(i, carry):
                cnt_v[pl.ds(i * 16, 16)] = jnp.zeros((16,), jnp.float32)
                return carry
            lax.fori_loop(0, _CNT_PAD // 16, zbody, 0)

        plsc.subcore_barrier()

        ones16 = jnp.full((16,), 1.0, jnp.float32)

        def _counts(j):
            for g in range(chunk // 16):
                idx = dst_v[j, pl.ds(g * 16, 16)]
                plsc.addupdate_scatter(cnt_v, [idx], ones16)

        def _gather(j, rows, sem):
            pltpu.async_copy(h_hbm.at[src_v.at[j]], rows, sem)

        def _wait(j, rows, sem):
            pltpu.make_async_copy(h_hbm.at[src_v.at[j]], rows, sem).wait()

        def _scatter_start(j, rows, sem):
            d = pltpu.async_copy(rows, agg_sh.at[dst_v.at[j]], sem, add=True)
            if with_counts:
                _counts(j)
            return d

        def _process(j, rows):
            _scatter_start(j, rows, sem_s).wait()

        def outer(b, carry):
            pltpu.sync_copy(src_hbm.at[wid, b], src_v)
            pltpu.sync_copy(dst_hbm.at[wid, b], dst_v)

            _gather(0, rows_a, sem_a)

            def pair(t, c2):
                j0 = 2 * t
                _gather(j0 + 1, rows_b, sem_b)
                _wait(j0, rows_a, sem_a)
                da = _scatter_start(j0, rows_a, sem_s)
                _wait(j0 + 1, rows_b, sem_b)
                db = _scatter_start(j0 + 1, rows_b, sem_t)
                da.wait()
                _gather(j0 + 2, rows_a, sem_a)
                db.wait()
                return c2

            if idxblk % 2:
                carry = lax.fori_loop(0, (idxblk - 1) // 2, pair, carry)
                jl = idxblk - 1
                _wait(jl, rows_a, sem_a)
                _process(jl, rows_a)
            else:
                carry = lax.fori_loop(0, idxblk // 2 - 1, pair, carry)
                j0 = idxblk - 2
                _gather(j0 + 1, rows_b, sem_b)
                _wait(j0, rows_a, sem_a)
                da = _scatter_start(j0, rows_a, sem_s)
                _wait(j0 + 1, rows_b, sem_b)
                db = _scatter_start(j0 + 1, rows_b, sem_t)
                da.wait()
                db.wait()
            return carry

        lax.fori_loop(0, nidx, outer, 0)

        plsc.subcore_barrier()

        pltpu.sync_copy(agg_sh.at[pl.ds(zbase, _SLAB)],
                        agg_out.at[cid, pl.ds(zbase, _SLAB)])
        if with_counts:
            for s in range(_NSLAB):
                pltpu.sync_copy(cnt_v.at[pl.ds(s * _BLK, _BLK)],
                                cnt_out.at[s, wid])

    return _sc_agg


_CH1, _IB1, _NX1 = 80, 25, 5
_CH2, _IB2, _NX2 = 125, 16, 5
_sc_agg1 = _make_sc_agg(_CH1, _IB1, _NX1, with_counts=True)
_sc_agg2 = _make_sc_agg(_CH2, _IB2, _NX2, with_counts=False)


def _dotg(a, w):
    return lax.dot_general(a, w, (((1,), (1,)), ((), ())),
                           preferred_element_type=jnp.float32)


def _mean(aggp_ref, cntp_ref):
    agg = aggp_ref[0] + aggp_ref[1]
    cnt = jnp.sum(cntp_ref[0], axis=0)[:, None]
    return agg / jnp.maximum(cnt, 1.0)


def _tc1_body(aggp_ref, cntp_ref, x_ref, wrel_ref, brel_ref, wroot_ref, out_ref):
    agg = _mean(aggp_ref, cntp_ref)
    h = _dotg(agg, wrel_ref[...]) + brel_ref[...] + _dotg(x_ref[...], wroot_ref[...])
    out_ref[...] = _lrelu(h)


def _tc2_body(aggp_ref, cntp_ref, h1_ref, wrel_ref, brel_ref, wroot_ref,
              wl0_ref, bl0_ref, wl1_ref, bl1_ref, wl2_ref, bl2_ref, out_ref):
    agg = _mean(aggp_ref, cntp_ref)
    h2 = _dotg(agg, wrel_ref[...]) + brel_ref[...] + _dotg(h1_ref[...], wroot_ref[...])
    t = _lrelu(_dotg(h2, wl0_ref[...]) + bl0_ref[...])
    t = _lrelu(_dotg(t, wl1_ref[...]) + bl1_ref[...])
    y = jnp.sum(t * wl2_ref[...], axis=1, keepdims=True) + bl2_ref[0, 0]
    out_ref[...] = jax.nn.sigmoid(y)


_full128 = pl.BlockSpec((_D, _D), lambda i: (0, 0))
_bias = pl.BlockSpec((1, _D), lambda i: (0, 0))
_rows = pl.BlockSpec((_BLK, _D), lambda i: (i, 0))
_aggp_spec = pl.BlockSpec((_NC, _BLK, _D), lambda i: (0, i, 0))
_cntp_spec = pl.BlockSpec((1, _NW, _BLK), lambda i: (i, 0, 0))

_tc1 = pl.pallas_call(
    _tc1_body,
    grid=(_GRID,),
    in_specs=[_aggp_spec, _cntp_spec, _rows, _full128, _bias, _full128],
    out_specs=_rows,
    out_shape=jax.ShapeDtypeStruct((_N, _D), jnp.float32),
)

_tc2 = pl.pallas_call(
    _tc2_body,
    grid=(_GRID,),
    in_specs=[_aggp_spec, _cntp_spec, _rows, _full128, _bias, _full128,
              _full128, _bias, _full128, _bias,
              pl.BlockSpec((1, _D), lambda i: (0, 0)),
              pl.BlockSpec((1, 1), lambda i: (0, 0))],
    out_specs=pl.BlockSpec((_BLK, 1), lambda i: (i, 0)),
    out_shape=jax.ShapeDtypeStruct((_N, 1), jnp.float32),
)


def kernel(x, edge_index, W_rel1, b_rel1, W_root1, W_rel2, b_rel2, W_root2,
           W_l0, b_l0, W_l1, b_l1, W_l2, b_l2):
    src1 = edge_index[0].reshape(_NW, _NX1, _IB1, _CH1)
    dst1 = edge_index[1].reshape(_NW, _NX1, _IB1, _CH1)
    src2 = edge_index[0].reshape(_NW, _NX2, _IB2, _CH2)
    dst2 = edge_index[1].reshape(_NW, _NX2, _IB2, _CH2)

    agg1, cnt1 = _sc_agg1(x, src1, dst1)
    h1 = _tc1(agg1, cnt1, x, W_rel1, b_rel1.reshape(1, _D), W_root1)
    res2 = _sc_agg2(h1, src2, dst2)
    agg2 = res2[0] if isinstance(res2, (tuple, list)) else res2
    out = _tc2(agg2, cnt1, h1, W_rel2, b_rel2.reshape(1, _D), W_root2,
               W_l0, b_l0.reshape(1, _D), W_l1, b_l1.reshape(1, _D),
               W_l2, b_l2.reshape(1, 1))
    return out


# speedup vs baseline: 1.2541x; 1.2541x over previous
import functools

import jax
import jax.numpy as jnp
from jax import lax
from jax.experimental import pallas as pl
from jax.experimental.pallas import tpu as pltpu
from jax.experimental.pallas import tpu_sc as plsc

_N = 10000
_E = 320000
_D = 128

_NC = 2
_NS = 16
_NW = _NC * _NS

_EPT = _E // _NW
_N_PAD = 10112
_SLAB = _N_PAD // _NS

_BLK = 1280
_GRID = 8
_NSLAB = 8
_CNT_PAD = _NSLAB * _BLK


def _lrelu(v):
    return jnp.where(v >= 0, v, 0.01 * v)


_sc_mesh = plsc.VectorSubcoreMesh(core_axis_name="c", subcore_axis_name="s")


def _make_sc_agg(chunk, idxblk, nidx, with_counts):
    out_type = [jax.ShapeDtypeStruct((_NC, _N_PAD, _D), jnp.float32)]
    scratch = [
        pltpu.VMEM((idxblk, chunk), jnp.int32),
        pltpu.VMEM((idxblk, chunk), jnp.int32),
        pltpu.VMEM((chunk, _D), jnp.float32),
        pltpu.VMEM((chunk, _D), jnp.float32),
        pltpu.VMEM_SHARED((_N_PAD, _D), jnp.float32),
        pltpu.SemaphoreType.DMA,
        pltpu.SemaphoreType.DMA,
        pltpu.SemaphoreType.DMA,
        pltpu.SemaphoreType.DMA,
    ]
    if with_counts:
        out_type.append(jax.ShapeDtypeStruct((_NSLAB, _NW, _BLK), jnp.float32))
        scratch.insert(4, pltpu.VMEM((_CNT_PAD,), jnp.float32))

    @functools.partial(
        pl.kernel,
        mesh=_sc_mesh,
        compiler_params=pltpu.CompilerParams(needs_layout_passes=False),
        out_type=tuple(out_type),
        scratch_types=scratch,
    )
    def _sc_agg(h_hbm, edge_hbm, *rest):
        if with_counts:
            (agg_out, cnt_out, src_v, dst_v, rows_a, rows_b, cnt_v,
             agg_sh, sem_a, sem_b, sem_s, sem_t) = rest
        else:
            (agg_out, src_v, dst_v, rows_a, rows_b,
             agg_sh, sem_a, sem_b, sem_s, sem_t) = rest
        cid = lax.axis_index("c")
        sid = lax.axis_index("s")
        wid = sid * _NC + cid

        zbase = sid * _SLAB
        def zrow(i, carry):
            for g in range(_D // 16):
                rows_a[i, pl.ds(g * 16, 16)] = jnp.zeros((16,), jnp.float32)
            return carry
        lax.fori_loop(0, chunk, zrow, 0)
        for f in range(_SLAB // chunk):
            pltpu.sync_copy(rows_a, agg_sh.at[pl.ds(zbase + f * chunk, chunk)])
        rem = _SLAB % chunk
        if rem:
            pltpu.sync_copy(
                rows_a.at[pl.ds(0, rem)],
                agg_sh.at[pl.ds(zbase + (_SLAB // chunk) * chunk, rem)])

        if with_counts:
            def zbody(i, carry):
                cnt_v[pl.ds(i * 16, 16)] = jnp.zeros((16,), jnp.float32)
                return carry
            lax.fori_loop(0, _CNT_PAD // 16, zbody, 0)

        plsc.subcore_barrier()

        ones16 = jnp.full((16,), 1.0, jnp.float32)

        def _counts(j):
            for g in range(chunk // 16):
                idx = dst_v[j, pl.ds(g * 16, 16)]
                plsc.addupdate_scatter(cnt_v, [idx], ones16)

        def _gather(j, rows, sem):
            pltpu.async_copy(h_hbm.at[src_v.at[j]], rows, sem)

        def _wait(j, rows, sem):
            pltpu.make_async_copy(h_hbm.at[src_v.at[j]], rows, sem).wait()

        def _scatter_start(j, rows, sem):
            d = pltpu.async_copy(rows, agg_sh.at[dst_v.at[j]], sem, add=True)
            if with_counts:
                _counts(j)
            return d

        def _process(j, rows):
            _scatter_start(j, rows, sem_s).wait()

        def outer(b, carry):
            pltpu.sync_copy(edge_hbm.at[0, wid, b], src_v)
            pltpu.sync_copy(edge_hbm.at[1, wid, b], dst_v)

            _gather(0, rows_a, sem_a)

            def pair(t, c2):
                j0 = 2 * t
                _gather(j0 + 1, rows_b, sem_b)
                _wait(j0, rows_a, sem_a)
                _process(j0, rows_a)
                _gather(j0 + 2, rows_a, sem_a)
                _wait(j0 + 1, rows_b, sem_b)
                _process(j0 + 1, rows_b)
                return c2

            if idxblk % 2:
                carry = lax.fori_loop(0, (idxblk - 1) // 2, pair, carry)
                jl = idxblk - 1
                _wait(jl, rows_a, sem_a)
                _process(jl, rows_a)
            else:
                carry = lax.fori_loop(0, idxblk // 2 - 1, pair, carry)
                j0 = idxblk - 2
                _gather(j0 + 1, rows_b, sem_b)
                _wait(j0, rows_a, sem_a)
                _process(j0, rows_a)
                _wait(j0 + 1, rows_b, sem_b)
                _process(j0 + 1, rows_b)
            return carry

        lax.fori_loop(0, nidx, outer, 0)

        plsc.subcore_barrier()

        pltpu.sync_copy(agg_sh.at[pl.ds(zbase, _SLAB)],
                        agg_out.at[cid, pl.ds(zbase, _SLAB)])
        if with_counts:
            for s in range(_NSLAB):
                pltpu.sync_copy(cnt_v.at[pl.ds(s * _BLK, _BLK)],
                                cnt_out.at[s, wid])

    return _sc_agg


_CH1, _IB1, _NX1 = 80, 25, 5
_CH2, _IB2, _NX2 = 125, 16, 5
_sc_agg1 = _make_sc_agg(_CH1, _IB1, _NX1, with_counts=True)
_sc_agg2 = _make_sc_agg(_CH2, _IB2, _NX2, with_counts=False)


def _dotg(a, w):
    return lax.dot_general(a, w, (((1,), (1,)), ((), ())),
                           preferred_element_type=jnp.float32)


def _mean(aggp_ref, cntp_ref):
    agg = aggp_ref[0] + aggp_ref[1]
    cnt = jnp.sum(cntp_ref[0], axis=0)[:, None]
    return agg / jnp.maximum(cnt, 1.0)


def _tc_root_body(x_ref, wroot_ref, out_ref):
    out_ref[...] = _dotg(x_ref[...], wroot_ref[...])


def _tc1_body(aggp_ref, cntp_ref, xroot_ref, wrel_ref, brel_ref, out_ref):
    agg = _mean(aggp_ref, cntp_ref)
    h = _dotg(agg, wrel_ref[...]) + brel_ref[...] + xroot_ref[...]
    out_ref[...] = _lrelu(h)


def _tc2_body(aggp_ref, cntp_ref, hroot_ref, wrel_ref, brel_ref,
              wl0_ref, bl0_ref, wl1_ref, bl1_ref, wl2_ref, bl2_ref, out_ref):
    agg = _mean(aggp_ref, cntp_ref)
    h2 = _dotg(agg, wrel_ref[...]) + brel_ref[...] + hroot_ref[...]
    t = _lrelu(_dotg(h2, wl0_ref[...]) + bl0_ref[...])
    t = _lrelu(_dotg(t, wl1_ref[...]) + bl1_ref[...])
    y = jnp.sum(t * wl2_ref[...], axis=1, keepdims=True) + bl2_ref[0, 0]
    out_ref[...] = jax.nn.sigmoid(y)


_full128 = pl.BlockSpec((_D, _D), lambda i: (0, 0))
_bias = pl.BlockSpec((1, _D), lambda i: (0, 0))
_rows = pl.BlockSpec((_BLK, _D), lambda i: (i, 0))
_aggp_spec = pl.BlockSpec((_NC, _BLK, _D), lambda i: (0, i, 0))
_cntp_spec = pl.BlockSpec((1, _NW, _BLK), lambda i: (i, 0, 0))

_tc_root = pl.pallas_call(
    _tc_root_body,
    grid=(_GRID,),
    in_specs=[_rows, _full128],
    out_specs=_rows,
    out_shape=jax.ShapeDtypeStruct((_N, _D), jnp.float32),
)

_tc1 = pl.pallas_call(
    _tc1_body,
    grid=(_GRID,),
    in_specs=[_aggp_spec, _cntp_spec, _rows, _full128, _bias],
    out_specs=_rows,
    out_shape=jax.ShapeDtypeStruct((_N, _D), jnp.float32),
)

_tc2 = pl.pallas_call(
    _tc2_body,
    grid=(_GRID,),
    in_specs=[_aggp_spec, _cntp_spec, _rows, _full128, _bias,
              _full128, _bias, _full128, _bias,
              pl.BlockSpec((1, _D), lambda i: (0, 0)),
              pl.BlockSpec((1, 1), lambda i: (0, 0))],
    out_specs=pl.BlockSpec((_BLK, 1), lambda i: (i, 0)),
    out_shape=jax.ShapeDtypeStruct((_N, 1), jnp.float32),
)


def kernel(x, edge_index, W_rel1, b_rel1, W_root1, W_rel2, b_rel2, W_root2,
           W_l0, b_l0, W_l1, b_l1, W_l2, b_l2):
    edges1 = edge_index.reshape(2, _NW, _NX1, _IB1, _CH1)
    edges2 = edge_index.reshape(2, _NW, _NX2, _IB2, _CH2)

    xroot = _tc_root(x, W_root1)
    agg1, cnt1 = _sc_agg1(x, edges1)
    h1 = _tc1(agg1, cnt1, xroot, W_rel1, b_rel1.reshape(1, _D))
    hroot = _tc_root(h1, W_root2)
    res2 = _sc_agg2(h1, edges2)
    agg2 = res2[0] if isinstance(res2, (tuple, list)) else res2
    out = _tc2(agg2, cnt1, hroot, W_rel2, b_rel2.reshape(1, _D),
               W_l0, b_l0.reshape(1, _D), W_l1, b_l1.reshape(1, _D),
               W_l2, b_l2.reshape(1, 1))
    return out
